# baseline (speedup 1.0000x reference)
import functools

import jax
import jax.numpy as jnp
from jax import lax
from jax.experimental import pallas as pl
from jax.experimental.pallas import tpu as pltpu
from jax.experimental.pallas import tpu_sc as plsc

_E = 8
_TOPK = 2
_TILE = 256


def _router_body(xr_ref, wg_ref, bg_ref, eb_ref, idx_ref, w_ref):
    logits = jax.lax.dot_general(
        xr_ref[...], wg_ref[...],
        dimension_numbers=(((1,), (1,)), ((), ())),
        preferred_element_type=jnp.float32,
    ) + bg_ref[0]
    s = logits + eb_ref[0]
    col = jax.lax.broadcasted_iota(jnp.int32, s.shape, 1)
    m1 = jnp.max(s, axis=1, keepdims=True)
    i1 = jnp.min(jnp.where(s == m1, col, _E), axis=1)
    s2 = jnp.where(col == i1[:, None], -jnp.inf, s)
    m2 = jnp.max(s2, axis=1, keepdims=True)
    i2 = jnp.min(jnp.where(s2 == m2, col, _E), axis=1)
    g1 = jnp.sum(jnp.where(col == i1[:, None], logits, 0.0), axis=1)
    g2 = jnp.sum(jnp.where(col == i2[:, None], logits, 0.0), axis=1)
    mx = jnp.maximum(g1, g2)
    e1 = jnp.exp(g1 - mx)
    e2 = jnp.exp(g2 - mx)
    tot = e1 + e2
    col2i = jax.lax.broadcasted_iota(jnp.int32, idx_ref.shape, 1)
    idx_ref[...] = jnp.where(col2i == 0, i1[:, None], i2[:, None])
    w_ref[...] = jnp.where(col2i == 0, (e1 / tot)[:, None], (e2 / tot)[:, None])


def _ffn_body(te_ref, x_ref, w1_ref, b1_ref, w2_ref, b2_ref, w3_ref, b3_ref,
              g_ref, y_ref):
    x = x_ref[...]
    h = jax.lax.dot_general(
        x, w1_ref[0], (((1,), (1,)), ((), ())),
        preferred_element_type=jnp.float32) + b1_ref[0, 0]
    h = jnp.maximum(h, 0.0)
    h = jax.lax.dot_general(
        h, w2_ref[0], (((1,), (1,)), ((), ())),
        preferred_element_type=jnp.float32) + b2_ref[0, 0]
    h = jnp.maximum(h, 0.0)
    o = jax.lax.dot_general(
        h, w3_ref[0], (((1,), (1,)), ((), ())),
        preferred_element_type=jnp.float32) + b3_ref[0, 0]
    y_ref[...] = o * g_ref[0, 0][:, None]


def _make_sc_combine(P, B, D):
    info = plsc.get_sparse_core_info()
    NW = info.num_cores * info.num_subcores
    NC = info.num_cores
    bw = B // NW
    C = 32
    NCH = bw // C

    mesh = plsc.VectorSubcoreMesh(core_axis_name="c", subcore_axis_name="s")

    @functools.partial(
        pl.kernel, mesh=mesh,
        out_type=jax.ShapeDtypeStruct((B, D), jnp.float32),
        scratch_types=[
            pltpu.VMEM((bw,), jnp.int32),
            pltpu.VMEM((bw,), jnp.int32),
            pltpu.VMEM((C, D), jnp.float32),
            pltpu.VMEM((C, D), jnp.float32),
            pltpu.SemaphoreType.DMA,
            pltpu.SemaphoreType.DMA,
        ],
    )
    def k(y_hbm, p0_hbm, p1_hbm, out_hbm, i0_v, i1_v, bufa, bufb, sema, semb):
        wid = lax.axis_index("s") * NC + lax.axis_index("c")
        base = wid * bw
        pltpu.sync_copy(p0_hbm.at[pl.ds(base, bw)], i0_v)
        pltpu.sync_copy(p1_hbm.at[pl.ds(base, bw)], i1_v)

        def chunk(ci, carry):
            cpa = pltpu.async_copy(
                y_hbm.at[i0_v.at[pl.ds(ci * C, C)]], bufa, sema)
            cpb = pltpu.async_copy(
                y_hbm.at[i1_v.at[pl.ds(ci * C, C)]], bufb, semb)
            cpa.wait()
            cpb.wait()

            def row(r, carry2):
                for q in range(D // 16):
                    bufa[r, pl.ds(q * 16, 16)] = (
                        bufa[r, pl.ds(q * 16, 16)] + bufb[r, pl.ds(q * 16, 16)]
                    )
                return carry2

            lax.fori_loop(0, C, row, 0, unroll=False)
            pltpu.sync_copy(bufa, out_hbm.at[pl.ds(base + ci * C, C)])
            return carry

        lax.fori_loop(0, NCH, chunk, 0, unroll=False)

    return k


def _make_sc_dispatch(B, P, D):
    info = plsc.get_sparse_core_info()
    NW = info.num_cores * info.num_subcores
    NC = info.num_cores
    pw = P // NW
    C = 32
    NCH = pw // C

    mesh = plsc.VectorSubcoreMesh(core_axis_name="c", subcore_axis_name="s")

    @functools.partial(
        pl.kernel, mesh=mesh,
        out_type=jax.ShapeDtypeStruct((P, D), jnp.float32),
        scratch_types=[
            pltpu.VMEM((pw,), jnp.int32),
            pltpu.VMEM((C, D), jnp.float32),
            pltpu.VMEM((C, D), jnp.float32),
            pltpu.SemaphoreType.DMA,
            pltpu.SemaphoreType.DMA,
        ],
    )
    def k(x_hbm, idx_hbm, out_hbm, idx_v, bufa, bufb, sema, semb):
        wid = lax.axis_index("s") * NC + lax.axis_index("c")
        base = wid * pw
        pltpu.sync_copy(idx_hbm.at[pl.ds(base, pw)], idx_v)

        def chunk(ci, carry):
            buf, sem = bufa, sema
            cp = pltpu.async_copy(
                x_hbm.at[idx_v.at[pl.ds(ci * C, C)]], buf, sem)
            cp.wait()
            pltpu.sync_copy(buf, out_hbm.at[pl.ds(base + ci * C, C)])
            return carry

        lax.fori_loop(0, NCH, chunk, 0, unroll=False)

    return k


@jax.jit
def kernel(x_router_input, x_expert_input, Wg, bg, expert_biases,
           W1, b1, W2, b2, W3, b3):
    B, D_R = x_router_input.shape
    _, D_IN = x_expert_input.shape
    E, H1, _ = W1.shape
    _, H2, _ = W2.shape
    _, D_OUT, _ = W3.shape
    T = _TILE
    N = B * _TOPK
    P = N + E * T
    NT = P // T

    BM = 512
    top_k_indices, gw = pl.pallas_call(
        _router_body,
        grid=(B // BM,),
        in_specs=[
            pl.BlockSpec((BM, D_R), lambda i: (i, 0)),
            pl.BlockSpec((E, D_R), lambda i: (0, 0)),
            pl.BlockSpec((1, E), lambda i: (0, 0)),
            pl.BlockSpec((1, E), lambda i: (0, 0)),
        ],
        out_specs=[
            pl.BlockSpec((BM, _TOPK), lambda i: (i, 0)),
            pl.BlockSpec((BM, _TOPK), lambda i: (i, 0)),
        ],
        out_shape=[
            jax.ShapeDtypeStruct((B, _TOPK), jnp.int32),
            jax.ShapeDtypeStruct((B, _TOPK), jnp.float32),
        ],
    )(x_router_input, Wg, bg.reshape(1, E), expert_biases.reshape(1, E))

    flat_e = top_k_indices.reshape(-1)
    iota_n = jnp.arange(N, dtype=jnp.int32)
    skey = jnp.sort(flat_e * N + iota_n)
    slot_sorted = skey % N
    e_sorted = skey // N
    counts = jnp.sum(flat_e[:, None] == jnp.arange(E)[None, :],
                     axis=0).astype(jnp.int32)
    pc = ((counts + T - 1) // T) * T
    ends = jnp.cumsum(pc)
    starts = ends - pc
    starts_u = jnp.cumsum(counts) - counts
    p_j = jnp.arange(N, dtype=jnp.int32) + (starts - starts_u)[e_sorted]
    _, pos = jax.lax.sort((slot_sorted, p_j), num_keys=1)
    pp = jnp.arange(P, dtype=jnp.int32)
    e_p = jnp.minimum(jnp.sum(pp[:, None] >= ends[None, :], axis=1),
                      E - 1).astype(jnp.int32)
    jj = pp - (starts - starts_u)[e_p]
    valid = (pp - starts[e_p]) < counts[e_p]
    islot_c = jnp.where(valid, slot_sorted[jnp.clip(jj, 0, N - 1)], 0)
    src_row = islot_c // _TOPK
    gates = jnp.where(valid, gw.reshape(-1)[islot_c], 0.0)
    tile_expert = e_p[::T]

    xg = _make_sc_dispatch(B, P, D_IN)(x_expert_input, src_row)
    gates3 = gates.reshape(NT, 1, T)
    y = pl.pallas_call(
        _ffn_body,
        grid_spec=pltpu.PrefetchScalarGridSpec(
            num_scalar_prefetch=1,
            grid=(NT,),
            in_specs=[
                pl.BlockSpec((T, D_IN), lambda i, te: (i, 0)),
                pl.BlockSpec((1, H1, D_IN), lambda i, te: (te[i], 0, 0)),
                pl.BlockSpec((1, 1, H1), lambda i, te: (te[i], 0, 0)),
                pl.BlockSpec((1, H2, H1), lambda i, te: (te[i], 0, 0)),
                pl.BlockSpec((1, 1, H2), lambda i, te: (te[i], 0, 0)),
                pl.BlockSpec((1, D_OUT, H2), lambda i, te: (te[i], 0, 0)),
                pl.BlockSpec((1, 1, D_OUT), lambda i, te: (te[i], 0, 0)),
                pl.BlockSpec((1, 1, T), lambda i, te: (i, 0, 0)),
            ],
            out_specs=pl.BlockSpec((T, D_OUT), lambda i, te: (i, 0)),
        ),
        out_shape=jax.ShapeDtypeStruct((P, D_OUT), jnp.float32),
    )(tile_expert, xg, W1, b1.reshape(E, 1, H1), W2, b2.reshape(E, 1, H2),
      W3, b3.reshape(E, 1, D_OUT), gates3)

    pos2 = pos.reshape(B, _TOPK)
    final = _make_sc_combine(P, B, D_OUT)(
        y, pos2[:, 0], pos2[:, 1])
    return (final, top_k_indices)


# speedup vs baseline: 1.1119x; 1.1119x over previous
import functools

import jax
import jax.numpy as jnp
from jax import lax
from jax.experimental import pallas as pl
from jax.experimental.pallas import tpu as pltpu
from jax.experimental.pallas import tpu_sc as plsc

_E = 8
_TOPK = 2
_TILE = 256


def _router_body(xr_ref, wg_ref, bg_ref, eb_ref, idx_ref, w_ref):
    logits = jax.lax.dot_general(
        xr_ref[...], wg_ref[...],
        dimension_numbers=(((1,), (1,)), ((), ())),
        preferred_element_type=jnp.float32,
    ) + bg_ref[0]
    s = logits + eb_ref[0]
    col = jax.lax.broadcasted_iota(jnp.int32, s.shape, 1)
    m1 = jnp.max(s, axis=1, keepdims=True)
    i1 = jnp.min(jnp.where(s == m1, col, _E), axis=1)
    s2 = jnp.where(col == i1[:, None], -jnp.inf, s)
    m2 = jnp.max(s2, axis=1, keepdims=True)
    i2 = jnp.min(jnp.where(s2 == m2, col, _E), axis=1)
    g1 = jnp.sum(jnp.where(col == i1[:, None], logits, 0.0), axis=1)
    g2 = jnp.sum(jnp.where(col == i2[:, None], logits, 0.0), axis=1)
    mx = jnp.maximum(g1, g2)
    e1 = jnp.exp(g1 - mx)
    e2 = jnp.exp(g2 - mx)
    tot = e1 + e2
    col2i = jax.lax.broadcasted_iota(jnp.int32, idx_ref.shape, 1)
    idx_ref[...] = jnp.where(col2i == 0, i1[:, None], i2[:, None])
    w_ref[...] = jnp.where(col2i == 0, (e1 / tot)[:, None], (e2 / tot)[:, None])


def _ffn_body(te_ref, x_ref, w1_ref, b1_ref, w2_ref, b2_ref, w3_ref, b3_ref,
              g_ref, y_ref):
    x = x_ref[...]
    h = jax.lax.dot_general(
        x, w1_ref[0], (((1,), (1,)), ((), ())),
        preferred_element_type=jnp.float32) + b1_ref[0, 0]
    h = jnp.maximum(h, 0.0)
    h = jax.lax.dot_general(
        h, w2_ref[0], (((1,), (1,)), ((), ())),
        preferred_element_type=jnp.float32) + b2_ref[0, 0]
    h = jnp.maximum(h, 0.0)
    o = jax.lax.dot_general(
        h, w3_ref[0], (((1,), (1,)), ((), ())),
        preferred_element_type=jnp.float32) + b3_ref[0, 0]
    y_ref[...] = o * g_ref[0, 0][:, None]


def _make_sc_combine(P, B, D):
    info = plsc.get_sparse_core_info()
    NW = info.num_cores * info.num_subcores
    NC = info.num_cores
    bw = B // NW
    C = 16
    NCH = bw // C

    mesh = plsc.VectorSubcoreMesh(core_axis_name="c", subcore_axis_name="s")

    @functools.partial(
        pl.kernel, mesh=mesh,
        out_type=jax.ShapeDtypeStruct((B, D), jnp.float32),
        scratch_types=[
            pltpu.VMEM((bw,), jnp.int32),
            pltpu.VMEM((bw,), jnp.int32),
            pltpu.VMEM((2, C, D), jnp.float32),
            pltpu.VMEM((2, C, D), jnp.float32),
            pltpu.SemaphoreType.DMA,
            pltpu.SemaphoreType.DMA,
            pltpu.SemaphoreType.DMA,
            pltpu.SemaphoreType.DMA,
        ],
    )
    def k(y_hbm, p0_hbm, p1_hbm, out_hbm, i0_v, i1_v, bufa, bufb,
          s0a, s0b, s1a, s1b):
        wid = lax.axis_index("s") * NC + lax.axis_index("c")
        base = wid * bw
        pltpu.sync_copy(p0_hbm.at[pl.ds(base, bw)], i0_v)
        pltpu.sync_copy(p1_hbm.at[pl.ds(base, bw)], i1_v)
        sems = ((s0a, s0b), (s1a, s1b))

        def issue(ci, slot):
            sa, sb = sems[slot]
            pltpu.async_copy(y_hbm.at[i0_v.at[pl.ds(ci * C, C)]],
                             bufa.at[slot], sa)
            pltpu.async_copy(y_hbm.at[i1_v.at[pl.ds(ci * C, C)]],
                             bufb.at[slot], sb)

        def drain(slot):
            sa, sb = sems[slot]
            pltpu.make_async_copy(y_hbm.at[i0_v.at[pl.ds(0, C)]],
                                  bufa.at[slot], sa).wait()
            pltpu.make_async_copy(y_hbm.at[i1_v.at[pl.ds(0, C)]],
                                  bufb.at[slot], sb).wait()

        issue(0, 0)
        for ci in range(NCH):
            slot = ci % 2
            if ci + 1 < NCH:
                issue(ci + 1, 1 - slot)
            drain(slot)

            def row(r, carry2, _slot=slot):
                for q in range(D // 16):
                    bufa[_slot, r, pl.ds(q * 16, 16)] = (
                        bufa[_slot, r, pl.ds(q * 16, 16)]
                        + bufb[_slot, r, pl.ds(q * 16, 16)]
                    )
                return carry2

            lax.fori_loop(0, C, row, 0, unroll=False)
            pltpu.sync_copy(bufa.at[slot],
                            out_hbm.at[pl.ds(base + ci * C, C)])

    return k


@jax.jit
def kernel(x_router_input, x_expert_input, Wg, bg, expert_biases,
           W1, b1, W2, b2, W3, b3):
    B, D_R = x_router_input.shape
    _, D_IN = x_expert_input.shape
    E, H1, _ = W1.shape
    _, H2, _ = W2.shape
    _, D_OUT, _ = W3.shape
    T = _TILE
    N = B * _TOPK
    P = N + E * T
    NT = P // T

    BM = 512
    top_k_indices, gw = pl.pallas_call(
        _router_body,
        grid=(B // BM,),
        in_specs=[
            pl.BlockSpec((BM, D_R), lambda i: (i, 0)),
            pl.BlockSpec((E, D_R), lambda i: (0, 0)),
            pl.BlockSpec((1, E), lambda i: (0, 0)),
            pl.BlockSpec((1, E), lambda i: (0, 0)),
        ],
        out_specs=[
            pl.BlockSpec((BM, _TOPK), lambda i: (i, 0)),
            pl.BlockSpec((BM, _TOPK), lambda i: (i, 0)),
        ],
        out_shape=[
            jax.ShapeDtypeStruct((B, _TOPK), jnp.int32),
            jax.ShapeDtypeStruct((B, _TOPK), jnp.float32),
        ],
    )(x_router_input, Wg, bg.reshape(1, E), expert_biases.reshape(1, E))

    flat_e = top_k_indices.reshape(-1)
    iota_n = jnp.arange(N, dtype=jnp.int32)
    skey = jnp.sort(flat_e * N + iota_n)
    slot_sorted = skey % N
    e_sorted = skey // N
    counts = jnp.sum(flat_e[:, None] == jnp.arange(E)[None, :],
                     axis=0).astype(jnp.int32)
    pc = ((counts + T - 1) // T) * T
    ends = jnp.cumsum(pc)
    starts = ends - pc
    starts_u = jnp.cumsum(counts) - counts
    p_j = jnp.arange(N, dtype=jnp.int32) + (starts - starts_u)[e_sorted]
    _, pos = jax.lax.sort((slot_sorted, p_j), num_keys=1)
    pp = jnp.arange(P, dtype=jnp.int32)
    e_p = jnp.minimum(jnp.sum(pp[:, None] >= ends[None, :], axis=1),
                      E - 1).astype(jnp.int32)
    jj = pp - (starts - starts_u)[e_p]
    valid = (pp - starts[e_p]) < counts[e_p]
    islot_c = jnp.where(valid, slot_sorted[jnp.clip(jj, 0, N - 1)], 0)
    src_row = islot_c // _TOPK
    gates = jnp.where(valid, gw.reshape(-1)[islot_c], 0.0)
    tile_expert = e_p[::T]

    xg = jnp.take(x_expert_input, src_row, axis=0)
    gates3 = gates.reshape(NT, 1, T)
    y = pl.pallas_call(
        _ffn_body,
        grid_spec=pltpu.PrefetchScalarGridSpec(
            num_scalar_prefetch=1,
            grid=(NT,),
            in_specs=[
                pl.BlockSpec((T, D_IN), lambda i, te: (i, 0)),
                pl.BlockSpec((1, H1, D_IN), lambda i, te: (te[i], 0, 0)),
                pl.BlockSpec((1, 1, H1), lambda i, te: (te[i], 0, 0)),
                pl.BlockSpec((1, H2, H1), lambda i, te: (te[i], 0, 0)),
                pl.BlockSpec((1, 1, H2), lambda i, te: (te[i], 0, 0)),
                pl.BlockSpec((1, D_OUT, H2), lambda i, te: (te[i], 0, 0)),
                pl.BlockSpec((1, 1, D_OUT), lambda i, te: (te[i], 0, 0)),
                pl.BlockSpec((1, 1, T), lambda i, te: (i, 0, 0)),
            ],
            out_specs=pl.BlockSpec((T, D_OUT), lambda i, te: (i, 0)),
        ),
        out_shape=jax.ShapeDtypeStruct((P, D_OUT), jnp.float32),
    )(tile_expert, xg, W1, b1.reshape(E, 1, H1), W2, b2.reshape(E, 1, H2),
      W3, b3.reshape(E, 1, D_OUT), gates3)

    pos2 = pos.reshape(B, _TOPK)
    final = _make_sc_combine(P, B, D_OUT)(
        y, pos2[:, 0], pos2[:, 1])
    return (final, top_k_indices)
